# Initial kernel scaffold; baseline (speedup 1.0000x reference)
import jax
import jax.numpy as jnp
from jax.experimental import pallas as pl


def kernel(contexts, targets, embedding_table):
    raise NotImplementedError("write your pallas kernel here")


# speedup vs baseline: 1.5185x; 1.5185x over previous
import functools

import jax
import jax.numpy as jnp
from jax import lax
from jax.experimental import pallas as pl
from jax.experimental.pallas import tpu as pltpu
from jax.experimental.pallas import tpu_sc as plsc

VOCAB = 1000
TOTAL = 1024 * 50

NUM_CORES = 2
NUM_SUBCORES = 16
NUM_WORKERS = NUM_CORES * NUM_SUBCORES
PER_W = TOTAL // NUM_WORKERS
CHUNK = 64
N_CHUNKS = PER_W // CHUNK


def _sc_body(table, idx_hbm, tgt_hbm, out_hbm, tlog_hbm,
             idx_v, tgt_v, rows_v, tlog_v, sem):
    wid = lax.axis_index("s") * NUM_CORES + lax.axis_index("c")
    base = wid * PER_W
    pltpu.sync_copy(idx_hbm.at[pl.ds(base, PER_W)], idx_v)
    pltpu.sync_copy(tgt_hbm.at[pl.ds(base, PER_W)], tgt_v)

    def chunk_body(c, carry):
        off = c * CHUNK
        pltpu.async_copy(table.at[idx_v.at[pl.ds(off, CHUNK)]], rows_v, sem).wait()
        pltpu.sync_copy(rows_v, out_hbm.at[pl.ds(base + off, CHUNK)])
        iota = lax.iota(jnp.int32, 16)
        for g in range(CHUNK // 16):
            rows_local = iota + g * 16
            cols = tgt_v[pl.ds(off + g * 16, 16)]
            tlog_v[pl.ds(off + g * 16, 16)] = plsc.load_gather(
                rows_v, [rows_local, cols])
        return carry

    lax.fori_loop(0, N_CHUNKS, chunk_body, 0)
    pltpu.sync_copy(tlog_v, tlog_hbm.at[pl.ds(base, PER_W)])


_sc_gather = functools.partial(
    pl.kernel,
    mesh=plsc.VectorSubcoreMesh(core_axis_name="c", subcore_axis_name="s"),
    out_type=[
        jax.ShapeDtypeStruct((TOTAL, VOCAB), jnp.float32),
        jax.ShapeDtypeStruct((TOTAL,), jnp.float32),
    ],
    scratch_types=[
        pltpu.VMEM((PER_W,), jnp.int32),
        pltpu.VMEM((PER_W,), jnp.int32),
        pltpu.VMEM((CHUNK, VOCAB), jnp.float32),
        pltpu.VMEM((PER_W,), jnp.float32),
        pltpu.SemaphoreType.DMA,
    ],
    compiler_params=pltpu.CompilerParams(
        use_tc_tiling_on_sc=False, needs_layout_passes=False),
)(_sc_body)


ROWS_BLK = 1024
N_BLKS = TOTAL // ROWS_BLK


def _loss_body(x_ref, t_ref, o_ref):
    g = pl.program_id(0)
    x = x_ref[...]
    m = jnp.max(x, axis=1, keepdims=True)
    s = jnp.sum(jnp.exp(x - m), axis=1, keepdims=True)
    lse = m + jnp.log(s)
    partial = jnp.sum(lse) - jnp.sum(t_ref[...])

    @pl.when(g == 0)
    def _init():
        o_ref[0, 0] = 0.0

    o_ref[0, 0] += partial

    @pl.when(g == N_BLKS - 1)
    def _fin():
        o_ref[0, 0] = o_ref[0, 0] / TOTAL


_loss_call = pl.pallas_call(
    _loss_body,
    grid=(N_BLKS,),
    in_specs=[
        pl.BlockSpec((ROWS_BLK, VOCAB), lambda g: (g, 0)),
        pl.BlockSpec((1, 1, ROWS_BLK), lambda g: (g, 0, 0)),
    ],
    out_specs=pl.BlockSpec(memory_space=pltpu.SMEM),
    out_shape=jax.ShapeDtypeStruct((1, 1), jnp.float32),
)


def kernel(contexts, targets, embedding_table):
    idx = contexts.reshape(-1).astype(jnp.int32)
    tgt = targets.reshape(-1).astype(jnp.int32)
    logits, tlog = _sc_gather(embedding_table, idx, tgt)
    loss = _loss_call(logits, tlog.reshape(N_BLKS, 1, ROWS_BLK))[0, 0]
    return logits, loss


# speedup vs baseline: 1.6991x; 1.1189x over previous
import functools

import jax
import jax.numpy as jnp
from jax import lax
from jax.experimental import pallas as pl
from jax.experimental.pallas import tpu as pltpu
from jax.experimental.pallas import tpu_sc as plsc

VOCAB = 1000
VOCAB_PAD = 1024
TOTAL = 1024 * 50

NUM_CORES = 2
NUM_SUBCORES = 16
NUM_WORKERS = NUM_CORES * NUM_SUBCORES
PER_W = TOTAL // NUM_WORKERS
CHUNK = 32
N_CHUNKS = PER_W // CHUNK


def _sc_body(table, idx_hbm, out_hbm, idx_v, rows_v, sem):
    wid = lax.axis_index("s") * NUM_CORES + lax.axis_index("c")
    base = wid * PER_W
    pltpu.sync_copy(idx_hbm.at[pl.ds(base, PER_W)], idx_v)

    def chunk_body(c, carry):
        off = c * CHUNK
        pltpu.async_copy(table.at[idx_v.at[pl.ds(off, CHUNK)]], rows_v, sem).wait()
        pltpu.sync_copy(rows_v, out_hbm.at[pl.ds(base + off, CHUNK)])
        return carry

    lax.fori_loop(0, N_CHUNKS, chunk_body, 0)


_sc_gather = functools.partial(
    pl.kernel,
    mesh=plsc.VectorSubcoreMesh(core_axis_name="c", subcore_axis_name="s"),
    out_type=[jax.ShapeDtypeStruct((TOTAL, VOCAB_PAD), jnp.float32)],
    scratch_types=[
        pltpu.VMEM((PER_W,), jnp.int32),
        pltpu.VMEM((CHUNK, VOCAB_PAD), jnp.float32),
        pltpu.SemaphoreType.DMA,
    ],
    compiler_params=pltpu.CompilerParams(use_tc_tiling_on_sc=True),
)(_sc_body)


ROWS_BLK = 512
N_BLKS = TOTAL // ROWS_BLK


def _loss_body(x_ref, t_ref, logits_ref, o_ref):
    g = pl.program_id(0)
    x = x_ref[...]
    logits_ref[...] = x[:, :VOCAB]
    col = lax.broadcasted_iota(jnp.int32, (ROWS_BLK, VOCAB_PAD), 1)
    xm = jnp.where(col < VOCAB, x, -jnp.inf)
    m = jnp.max(xm, axis=1, keepdims=True)
    s = jnp.sum(jnp.exp(xm - m), axis=1, keepdims=True)
    lse = m + jnp.log(s)
    tcol = jnp.transpose(t_ref[...].reshape(1, ROWS_BLK))
    tval = jnp.sum(jnp.where(col == tcol, x, 0.0), axis=1, keepdims=True)
    partial = jnp.sum(lse - tval)

    @pl.when(g == 0)
    def _init():
        o_ref[0, 0] = 0.0

    o_ref[0, 0] += partial

    @pl.when(g == N_BLKS - 1)
    def _fin():
        o_ref[0, 0] = o_ref[0, 0] / TOTAL


_loss_call = pl.pallas_call(
    _loss_body,
    grid=(N_BLKS,),
    in_specs=[
        pl.BlockSpec((ROWS_BLK, VOCAB_PAD), lambda g: (g, 0)),
        pl.BlockSpec((1, 1, ROWS_BLK), lambda g: (g, 0, 0)),
    ],
    out_specs=[
        pl.BlockSpec((ROWS_BLK, VOCAB), lambda g: (g, 0)),
        pl.BlockSpec(memory_space=pltpu.SMEM),
    ],
    out_shape=[
        jax.ShapeDtypeStruct((TOTAL, VOCAB), jnp.float32),
        jax.ShapeDtypeStruct((1, 1), jnp.float32),
    ],
)


def kernel(contexts, targets, embedding_table):
    idx = contexts.reshape(-1).astype(jnp.int32)
    tgt = targets.reshape(-1).astype(jnp.int32)
    table_pad = jnp.pad(embedding_table, ((0, 0), (0, VOCAB_PAD - VOCAB)))
    (inter,) = _sc_gather(table_pad, idx)
    logits, loss = _loss_call(inter, tgt.reshape(N_BLKS, 1, ROWS_BLK))
    return logits, loss[0, 0]


# speedup vs baseline: 2.9603x; 1.7423x over previous
import functools

import jax
import jax.numpy as jnp
from jax import lax
from jax.experimental import pallas as pl
from jax.experimental.pallas import tpu as pltpu
from jax.experimental.pallas import tpu_sc as plsc

VOCAB = 1000
VOCAB_PAD = 1024
TOTAL = 1024 * 50

NUM_CORES = 2
NUM_SUBCORES = 16
NUM_WORKERS = NUM_CORES * NUM_SUBCORES
ROWS_BLK = 1024
STAGE_BLKS = (25, 25)
N_STAGES = len(STAGE_BLKS)


def _make_sc_gather(per_w, chunk):
    n_chunks = per_w // chunk
    assert per_w % chunk == 0 and n_chunks % 2 == 0 and chunk % 8 == 0

    def _sc_body(table, idx_hbm, out_hbm, idx_v, rows_v0, rows_v1,
                 gsem0, gsem1, ssem0, ssem1):
        wid = lax.axis_index("s") * NUM_CORES + lax.axis_index("c")
        base = wid * per_w
        pltpu.sync_copy(idx_hbm.at[pl.ds(base, per_w)], idx_v)

        bufs = ((rows_v0, gsem0, ssem0), (rows_v1, gsem1, ssem1))

        for b, (rows_v, gsem, _) in enumerate(bufs):
            pltpu.async_copy(table.at[idx_v.at[pl.ds(b * chunk, chunk)]],
                             rows_v, gsem)

        def pair_body(i, carry):
            for b, (rows_v, gsem, ssem) in enumerate(bufs):
                c = 2 * i + b
                off = c * chunk
                pltpu.make_async_copy(
                    table.at[idx_v.at[pl.ds(off, chunk)]], rows_v, gsem).wait()
                pltpu.async_copy(rows_v,
                                 out_hbm.at[pl.ds(base + off, chunk)], ssem)

                @pl.when(c + 2 < n_chunks)
                def _next():
                    pltpu.make_async_copy(
                        rows_v, out_hbm.at[pl.ds(base + off, chunk)],
                        ssem).wait()
                    pltpu.async_copy(
                        table.at[idx_v.at[pl.ds(off + 2 * chunk, chunk)]],
                        rows_v, gsem)
            return carry

        lax.fori_loop(0, n_chunks // 2, pair_body, 0)

        for b, (rows_v, _, ssem) in enumerate(bufs):
            off = (n_chunks - 2 + b) * chunk
            pltpu.make_async_copy(
                rows_v, out_hbm.at[pl.ds(base + off, chunk)], ssem).wait()

    return functools.partial(
        pl.kernel,
        mesh=plsc.VectorSubcoreMesh(core_axis_name="c", subcore_axis_name="s"),
        out_type=[jax.ShapeDtypeStruct((per_w * NUM_WORKERS, VOCAB_PAD),
                                       jnp.float32)],
        scratch_types=[
            pltpu.VMEM((per_w,), jnp.int32),
            pltpu.VMEM((chunk, VOCAB_PAD), jnp.float32),
            pltpu.VMEM((chunk, VOCAB_PAD), jnp.float32),
            pltpu.SemaphoreType.DMA,
            pltpu.SemaphoreType.DMA,
            pltpu.SemaphoreType.DMA,
            pltpu.SemaphoreType.DMA,
        ],
        compiler_params=pltpu.CompilerParams(use_tc_tiling_on_sc=True),
    )(_sc_body)


def _stage_chunk(per_w):
    for chunk in (64, 56, 48, 40, 32, 24, 16, 8):
        if chunk * VOCAB_PAD * 4 * 2 > 420 * 1024:
            continue
        if per_w % chunk == 0 and (per_w // chunk) % 2 == 0:
            return chunk
    raise ValueError(per_w)


_sc_gathers = [
    _make_sc_gather(b * ROWS_BLK // NUM_WORKERS,
                    _stage_chunk(b * ROWS_BLK // NUM_WORKERS))
    for b in STAGE_BLKS
]


def _loss_stage_body(x_ref, t_ref, logits_ref, o_ref):
    g = pl.program_id(0)
    x = x_ref[...]
    xt = jnp.transpose(x)
    logits_ref[...] = xt[:VOCAB, :]
    row = lax.broadcasted_iota(jnp.int32, (VOCAB_PAD, ROWS_BLK), 0)
    xm = jnp.where(row < VOCAB, xt, -jnp.inf)
    m = jnp.max(xm, axis=0, keepdims=True)
    s = jnp.sum(jnp.exp(xm - m), axis=0, keepdims=True)
    lse = m + jnp.log(s)
    t = t_ref[...].reshape(1, ROWS_BLK)
    tval = jnp.sum(jnp.where(row == t, xt, 0.0), axis=0, keepdims=True)
    partial = jnp.sum(lse - tval)

    @pl.when(g == 0)
    def _init():
        o_ref[0, 0] = 0.0

    o_ref[0, 0] += partial


def _next_body(prev_ref, x_ref, t_ref, logits_ref, o_ref):
    del prev_ref
    _loss_stage_body(x_ref, t_ref, logits_ref, o_ref)


def _make_loss_calls():
    t_spec = pl.BlockSpec((1, 1, ROWS_BLK), lambda g: (g, 0, 0))
    out_shapes = [
        jax.ShapeDtypeStruct((VOCAB, TOTAL), jnp.float32),
        jax.ShapeDtypeStruct((1, 1), jnp.float32),
    ]
    calls, base = [], 0
    for s, nblk in enumerate(STAGE_BLKS):
        x_spec = pl.BlockSpec((ROWS_BLK, VOCAB_PAD), lambda g: (g, 0))
        o_spec = pl.BlockSpec(
            (VOCAB, ROWS_BLK), functools.partial(lambda b, g: (0, b + g), base))
        if s == 0:
            calls.append(pl.pallas_call(
                _loss_stage_body,
                grid=(nblk,),
                in_specs=[x_spec, t_spec],
                out_specs=[o_spec, pl.BlockSpec(memory_space=pltpu.SMEM)],
                out_shape=out_shapes,
            ))
        else:
            calls.append(pl.pallas_call(
                _next_body,
                grid=(nblk,),
                in_specs=[pl.BlockSpec(memory_space=pl.ANY), x_spec, t_spec],
                out_specs=[o_spec, pl.BlockSpec(memory_space=pltpu.SMEM)],
                out_shape=out_shapes,
                input_output_aliases={0: 0},
            ))
        base += nblk
    return calls


_loss_calls = _make_loss_calls()


def kernel(contexts, targets, embedding_table):
    idx = contexts.reshape(-1).astype(jnp.int32)
    tgt = targets.reshape(-1).astype(jnp.int32)
    table_pad = jnp.pad(embedding_table, ((0, 0), (0, VOCAB_PAD - VOCAB)))
    inters, row0 = [], 0
    for s, nblk in enumerate(STAGE_BLKS):
        rows = nblk * ROWS_BLK
        inters.append(_sc_gathers[s](table_pad, idx[row0:row0 + rows])[0])
        row0 += rows
    tgt3 = tgt.reshape(TOTAL // ROWS_BLK, 1, ROWS_BLK)
    blk0 = 0
    logits_t, partial = _loss_calls[0](
        inters[0], tgt3[blk0:blk0 + STAGE_BLKS[0]])
    partials = [partial]
    blk0 += STAGE_BLKS[0]
    for s in range(1, N_STAGES):
        logits_t, partial = _loss_calls[s](
            logits_t, inters[s], tgt3[blk0:blk0 + STAGE_BLKS[s]])
        partials.append(partial)
        blk0 += STAGE_BLKS[s]
    loss = sum(p[0, 0] for p in partials) / TOTAL
    return logits_t.T, loss
